# Initial kernel scaffold; baseline (speedup 1.0000x reference)
import jax
import jax.numpy as jnp
from jax.experimental import pallas as pl


def kernel(xyz, noise, params):
    raise NotImplementedError("write your pallas kernel here")


# speedup vs baseline: 1.0002x; 1.0002x over previous
import jax
import jax.numpy as jnp
from jax.experimental import pallas as pl

_SA_CFGS = [
    (1024, (0.05, 0.1), (16, 32), 9, ((16, 16, 32), (32, 32, 64))),
    (512, (0.1, 0.2), (16, 32), 96, ((64, 64, 128), (64, 96, 128))),
    (256, (0.2, 0.4), (16, 32), 256, ((128, 196, 256), (128, 196, 256))),
    (64, (0.4, 0.8), (16, 32), 512, ((256, 256, 4), (256, 384, 4))),
]


def _square_distance(src, dst):
    dist = -2.0 * jnp.einsum('bnc,bmc->bnm', src, dst)
    dist = dist + jnp.sum(src ** 2, -1)[:, :, None]
    dist = dist + jnp.sum(dst ** 2, -1)[:, None, :]
    return dist


def _index_points(points, idx):
    return jax.vmap(lambda p, i: p[i])(points, idx)


def _farthest_point_sample(xyz, npoint):
    B, N, _ = xyz.shape
    def step(carry, _):
        distance, farthest = carry
        centroid = jnp.take_along_axis(xyz, farthest[:, None, None], axis=1)
        dist = jnp.sum((xyz - centroid) ** 2, -1)
        distance = jnp.minimum(distance, dist)
        nxt = jnp.argmax(distance, axis=-1).astype(jnp.int32)
        return (distance, nxt), farthest
    init = (jnp.full((B, N), 1e10, dtype=jnp.float32), jnp.zeros((B,), dtype=jnp.int32))
    _, idxs = jax.lax.scan(step, init, None, length=npoint)
    return jnp.transpose(idxs, (1, 0))


def _query_ball_point(radius, nsample, xyz, new_xyz):
    B, N, _ = xyz.shape
    S = new_xyz.shape[1]
    sqrdists = _square_distance(new_xyz, xyz)
    group_idx = jnp.broadcast_to(jnp.arange(N, dtype=jnp.int32)[None, None, :], (B, S, N))
    group_idx = jnp.where(sqrdists > radius ** 2, N, group_idx)
    group_idx = jnp.sort(group_idx, axis=-1)[:, :, :nsample]
    group_first = group_idx[:, :, :1]
    group_idx = jnp.where(group_idx == N, group_first, group_idx)
    return group_idx


def _conv_bn_relu(x, W, b, gamma, beta):
    y = jnp.einsum('oc,bcks->boks', W, x) + b[None, :, None, None]
    mean = jnp.mean(y, axis=(0, 2, 3), keepdims=True)
    var = jnp.var(y, axis=(0, 2, 3), keepdims=True)
    y = (y - mean) / jnp.sqrt(var + 1e-5)
    y = gamma[None, :, None, None] * y + beta[None, :, None, None]
    return jax.nn.relu(y)


def _sa_msg(xyz_bcn, points_bcn, cfg, branch_params):
    npoint, radii, nsamples, in_ch, mlps = cfg
    xyz = jnp.transpose(xyz_bcn, (0, 2, 1))
    points = jnp.transpose(points_bcn, (0, 2, 1))
    fps_idx = _farthest_point_sample(jax.lax.stop_gradient(xyz), npoint)
    new_xyz = _index_points(xyz, fps_idx)
    outs = []
    for i in range(len(radii)):
        K = nsamples[i]
        gidx = _query_ball_point(radii[i], K, jax.lax.stop_gradient(xyz), jax.lax.stop_gradient(new_xyz))
        grouped_xyz = _index_points(xyz, gidx) - new_xyz[:, :, None, :]
        grouped_points = jnp.concatenate([_index_points(points, gidx), grouped_xyz], axis=-1)
        g = jnp.transpose(grouped_points, (0, 3, 2, 1))
        for (W, b, gamma, beta) in branch_params[i]:
            g = _conv_bn_relu(g, W, b, gamma, beta)
        outs.append(jnp.max(g, axis=2))
    return jnp.transpose(new_xyz, (0, 2, 1)), jnp.concatenate(outs, axis=1)


def _final_kernel(mean_ref, logv_ref, noise_ref, out_ref):
    logv = jnp.clip(logv_ref[...], -30.0, 20.0)
    stdev = jnp.sqrt(jnp.exp(logv))
    out_ref[...] = (mean_ref[...] + stdev * noise_ref[...]) * 0.18215


def kernel(xyz, noise, params):
    l_xyz = xyz[:, :3, :]
    l_points = xyz
    for cfg, bp in zip(_SA_CFGS, params):
        l_xyz, l_points = _sa_msg(l_xyz, l_points, cfg, bp)
    mean, log_variance = jnp.split(l_points, 2, axis=1)
    x = pl.pallas_call(
        _final_kernel,
        out_shape=jax.ShapeDtypeStruct(mean.shape, mean.dtype),
    )(mean, log_variance, noise)
    return x, l_xyz


# speedup vs baseline: 1.2970x; 1.2968x over previous
import functools

import jax
import jax.numpy as jnp
from jax.experimental import pallas as pl
from jax.experimental.pallas import tpu as pltpu


def _fps_body(npoint, xyz_ref, idx_ref, cx_ref, cy_ref, cz_ref, dist_ref):
    B = xyz_ref.shape[0]
    N = xyz_ref.shape[2]
    S = npoint
    x = xyz_ref[:, 0, :]
    y = xyz_ref[:, 1, :]
    z = xyz_ref[:, 2, :]
    dist_ref[...] = jnp.full((B, N), 1e10, dtype=jnp.float32)
    lane_n = jax.lax.broadcasted_iota(jnp.int32, (B, N), 1)
    lane_s = jax.lax.broadcasted_iota(jnp.int32, (B, S), 1)

    def step(t, far):
        sel_t = lane_s == t
        idx_ref[...] = jnp.where(sel_t, far, idx_ref[...])
        onehot = lane_n == far
        cx = jnp.sum(jnp.where(onehot, x, 0.0), axis=1, keepdims=True)
        cy = jnp.sum(jnp.where(onehot, y, 0.0), axis=1, keepdims=True)
        cz = jnp.sum(jnp.where(onehot, z, 0.0), axis=1, keepdims=True)
        cx_ref[...] = jnp.where(sel_t, cx, cx_ref[...])
        cy_ref[...] = jnp.where(sel_t, cy, cy_ref[...])
        cz_ref[...] = jnp.where(sel_t, cz, cz_ref[...])
        dx = x - cx
        dy = y - cy
        dz = z - cz
        d = (dx * dx + dy * dy) + dz * dz
        dnew = jnp.minimum(dist_ref[...], d)
        dist_ref[...] = dnew
        maxv = jnp.max(dnew, axis=1, keepdims=True)
        far_next = jnp.min(jnp.where(dnew == maxv, lane_n, N), axis=1,
                           keepdims=True).astype(jnp.int32)
        return far_next

    jax.lax.fori_loop(0, S, step, jnp.zeros((B, 1), jnp.int32))


def _fps_pallas(xyz_bcn, npoint):
    B, _, N = xyz_bcn.shape
    out_shape = [
        jax.ShapeDtypeStruct((B, npoint), jnp.int32),
        jax.ShapeDtypeStruct((B, npoint), jnp.float32),
        jax.ShapeDtypeStruct((B, npoint), jnp.float32),
        jax.ShapeDtypeStruct((B, npoint), jnp.float32),
    ]
    idx, cx, cy, cz = pl.pallas_call(
        functools.partial(_fps_body, npoint),
        out_shape=out_shape,
        scratch_shapes=[pltpu.VMEM((B, N), jnp.float32)],
    )(xyz_bcn)
    new_xyz = jnp.stack([cx, cy, cz], axis=-1)
    return idx, new_xyz

_SA_CFGS = [
    (1024, (0.05, 0.1), (16, 32), 9, ((16, 16, 32), (32, 32, 64))),
    (512, (0.1, 0.2), (16, 32), 96, ((64, 64, 128), (64, 96, 128))),
    (256, (0.2, 0.4), (16, 32), 256, ((128, 196, 256), (128, 196, 256))),
    (64, (0.4, 0.8), (16, 32), 512, ((256, 256, 4), (256, 384, 4))),
]


def _square_distance(src, dst):
    dist = -2.0 * jnp.einsum('bnc,bmc->bnm', src, dst)
    dist = dist + jnp.sum(src ** 2, -1)[:, :, None]
    dist = dist + jnp.sum(dst ** 2, -1)[:, None, :]
    return dist


def _index_points(points, idx):
    return jax.vmap(lambda p, i: p[i])(points, idx)


def _farthest_point_sample(xyz, npoint):
    B, N, _ = xyz.shape
    def step(carry, _):
        distance, farthest = carry
        centroid = jnp.take_along_axis(xyz, farthest[:, None, None], axis=1)
        dist = jnp.sum((xyz - centroid) ** 2, -1)
        distance = jnp.minimum(distance, dist)
        nxt = jnp.argmax(distance, axis=-1).astype(jnp.int32)
        return (distance, nxt), farthest
    init = (jnp.full((B, N), 1e10, dtype=jnp.float32), jnp.zeros((B,), dtype=jnp.int32))
    _, idxs = jax.lax.scan(step, init, None, length=npoint)
    return jnp.transpose(idxs, (1, 0))


def _query_ball_point(radius, nsample, xyz, new_xyz):
    B, N, _ = xyz.shape
    S = new_xyz.shape[1]
    sqrdists = _square_distance(new_xyz, xyz)
    group_idx = jnp.broadcast_to(jnp.arange(N, dtype=jnp.int32)[None, None, :], (B, S, N))
    group_idx = jnp.where(sqrdists > radius ** 2, N, group_idx)
    group_idx = jnp.sort(group_idx, axis=-1)[:, :, :nsample]
    group_first = group_idx[:, :, :1]
    group_idx = jnp.where(group_idx == N, group_first, group_idx)
    return group_idx


def _conv_bn_relu(x, W, b, gamma, beta):
    y = jnp.einsum('oc,bcks->boks', W, x) + b[None, :, None, None]
    mean = jnp.mean(y, axis=(0, 2, 3), keepdims=True)
    var = jnp.var(y, axis=(0, 2, 3), keepdims=True)
    y = (y - mean) / jnp.sqrt(var + 1e-5)
    y = gamma[None, :, None, None] * y + beta[None, :, None, None]
    return jax.nn.relu(y)


def _sa_msg(xyz_bcn, points_bcn, cfg, branch_params):
    npoint, radii, nsamples, in_ch, mlps = cfg
    xyz = jnp.transpose(xyz_bcn, (0, 2, 1))
    points = jnp.transpose(points_bcn, (0, 2, 1))
    fps_idx, new_xyz = _fps_pallas(xyz_bcn, npoint)
    outs = []
    for i in range(len(radii)):
        K = nsamples[i]
        gidx = _query_ball_point(radii[i], K, jax.lax.stop_gradient(xyz), jax.lax.stop_gradient(new_xyz))
        grouped_xyz = _index_points(xyz, gidx) - new_xyz[:, :, None, :]
        grouped_points = jnp.concatenate([_index_points(points, gidx), grouped_xyz], axis=-1)
        g = jnp.transpose(grouped_points, (0, 3, 2, 1))
        for (W, b, gamma, beta) in branch_params[i]:
            g = _conv_bn_relu(g, W, b, gamma, beta)
        outs.append(jnp.max(g, axis=2))
    return jnp.transpose(new_xyz, (0, 2, 1)), jnp.concatenate(outs, axis=1)


def _final_kernel(mean_ref, logv_ref, noise_ref, out_ref):
    logv = jnp.clip(logv_ref[...], -30.0, 20.0)
    stdev = jnp.sqrt(jnp.exp(logv))
    out_ref[...] = (mean_ref[...] + stdev * noise_ref[...]) * 0.18215


def kernel(xyz, noise, params):
    l_xyz = xyz[:, :3, :]
    l_points = xyz
    for cfg, bp in zip(_SA_CFGS, params):
        l_xyz, l_points = _sa_msg(l_xyz, l_points, cfg, bp)
    mean, log_variance = jnp.split(l_points, 2, axis=1)
    x = pl.pallas_call(
        _final_kernel,
        out_shape=jax.ShapeDtypeStruct(mean.shape, mean.dtype),
    )(mean, log_variance, noise)
    return x, l_xyz


# speedup vs baseline: 1.4726x; 1.1354x over previous
import functools

import jax
import jax.numpy as jnp
from jax.experimental import pallas as pl
from jax.experimental.pallas import tpu as pltpu


def _fps_body(npoint, xyz_ref, idx_ref, cx_ref, cy_ref, cz_ref, dist_ref):
    B = xyz_ref.shape[0]
    N = xyz_ref.shape[2]
    S = npoint
    x = xyz_ref[:, 0, :]
    y = xyz_ref[:, 1, :]
    z = xyz_ref[:, 2, :]
    dist_ref[...] = jnp.full((B, N), 1e10, dtype=jnp.float32)
    lane_n = jax.lax.broadcasted_iota(jnp.int32, (B, N), 1)
    lane_s = jax.lax.broadcasted_iota(jnp.int32, (B, S), 1)

    def step(t, far):
        sel_t = lane_s == t
        idx_ref[...] = jnp.where(sel_t, far, idx_ref[...])
        onehot = lane_n == far
        cx = jnp.sum(jnp.where(onehot, x, 0.0), axis=1, keepdims=True)
        cy = jnp.sum(jnp.where(onehot, y, 0.0), axis=1, keepdims=True)
        cz = jnp.sum(jnp.where(onehot, z, 0.0), axis=1, keepdims=True)
        cx_ref[...] = jnp.where(sel_t, cx, cx_ref[...])
        cy_ref[...] = jnp.where(sel_t, cy, cy_ref[...])
        cz_ref[...] = jnp.where(sel_t, cz, cz_ref[...])
        dx = x - cx
        dy = y - cy
        dz = z - cz
        d = (dx * dx + dy * dy) + dz * dz
        dnew = jnp.minimum(dist_ref[...], d)
        dist_ref[...] = dnew
        maxv = jnp.max(dnew, axis=1, keepdims=True)
        far_next = jnp.min(jnp.where(dnew == maxv, lane_n, N), axis=1,
                           keepdims=True).astype(jnp.int32)
        return far_next

    jax.lax.fori_loop(0, S, step, jnp.zeros((B, 1), jnp.int32))


def _fps_pallas(xyz_bcn, npoint):
    B, _, N = xyz_bcn.shape
    out_shape = [
        jax.ShapeDtypeStruct((B, npoint), jnp.int32),
        jax.ShapeDtypeStruct((B, npoint), jnp.float32),
        jax.ShapeDtypeStruct((B, npoint), jnp.float32),
        jax.ShapeDtypeStruct((B, npoint), jnp.float32),
    ]
    idx, cx, cy, cz = pl.pallas_call(
        functools.partial(_fps_body, npoint),
        out_shape=out_shape,
        scratch_shapes=[pltpu.VMEM((B, N), jnp.float32)],
    )(xyz_bcn)
    new_xyz = jnp.stack([cx, cy, cz], axis=-1)
    return idx, new_xyz

_SA_CFGS = [
    (1024, (0.05, 0.1), (16, 32), 9, ((16, 16, 32), (32, 32, 64))),
    (512, (0.1, 0.2), (16, 32), 96, ((64, 64, 128), (64, 96, 128))),
    (256, (0.2, 0.4), (16, 32), 256, ((128, 196, 256), (128, 196, 256))),
    (64, (0.4, 0.8), (16, 32), 512, ((256, 256, 4), (256, 384, 4))),
]


def _ballq_body(r2, K, N, Sblk, xyz_ref, nx_ref, gidx_ref):
    b = pl.program_id(0)
    si = pl.program_id(1)
    x = xyz_ref[b, 0:1, :]
    y = xyz_ref[b, 1:2, :]
    z = xyz_ref[b, 2:3, :]
    c = nx_ref[b, pl.ds(si * Sblk, Sblk), :]
    cx = c[:, 0:1]
    cy = c[:, 1:2]
    cz = c[:, 2:3]
    xmat = jnp.concatenate([x, y, z], axis=0).astype(jnp.bfloat16)
    dot = jax.lax.dot_general(
        c.astype(jnp.bfloat16), xmat, (((1,), (0,)), ((), ())),
        preferred_element_type=jnp.float32)
    csq = (cx * cx + cy * cy) + cz * cz
    xsq = (x * x + y * y) + z * z
    dist = (-2.0 * dot + csq) + xsq
    Sblk = c.shape[0]
    jf = jax.lax.broadcasted_iota(jnp.int32, (Sblk, N), 1).astype(jnp.float32)
    fN = jnp.float32(N)
    d0 = jnp.where(dist <= r2, jf, fN)
    cols = []
    for _ in range(K):
        m = jnp.min(d0, axis=1, keepdims=True)
        cols.append(m)
        d0 = jnp.where(d0 == m, fN, d0)
    out = jnp.concatenate(cols, axis=1)
    out = jnp.where(out == fN, cols[0], out)
    gidx_ref[b, pl.ds(si * Sblk, Sblk), :] = out.astype(jnp.int32)


def _ballq_pallas(radius, K, xyz_b3n, new_xyz):
    B, _, N = xyz_b3n.shape
    S = new_xyz.shape[1]
    Sblk = 8
    grid = (B, S // Sblk)
    return pl.pallas_call(
        functools.partial(_ballq_body, radius * radius, K, N, Sblk),
        grid=grid,
        in_specs=[
            pl.BlockSpec((B, 3, N), lambda b, s: (0, 0, 0)),
            pl.BlockSpec((B, S, 3), lambda b, s: (0, 0, 0)),
        ],
        out_specs=pl.BlockSpec((B, S, K), lambda b, s: (0, 0, 0)),
        out_shape=jax.ShapeDtypeStruct((B, S, K), jnp.int32),
    )(xyz_b3n, new_xyz)


def _square_distance(src, dst):
    dist = -2.0 * jnp.einsum('bnc,bmc->bnm', src, dst)
    dist = dist + jnp.sum(src ** 2, -1)[:, :, None]
    dist = dist + jnp.sum(dst ** 2, -1)[:, None, :]
    return dist


def _index_points(points, idx):
    return jax.vmap(lambda p, i: p[i])(points, idx)


def _farthest_point_sample(xyz, npoint):
    B, N, _ = xyz.shape
    def step(carry, _):
        distance, farthest = carry
        centroid = jnp.take_along_axis(xyz, farthest[:, None, None], axis=1)
        dist = jnp.sum((xyz - centroid) ** 2, -1)
        distance = jnp.minimum(distance, dist)
        nxt = jnp.argmax(distance, axis=-1).astype(jnp.int32)
        return (distance, nxt), farthest
    init = (jnp.full((B, N), 1e10, dtype=jnp.float32), jnp.zeros((B,), dtype=jnp.int32))
    _, idxs = jax.lax.scan(step, init, None, length=npoint)
    return jnp.transpose(idxs, (1, 0))


def _query_ball_point(radius, nsample, xyz, new_xyz):
    B, N, _ = xyz.shape
    S = new_xyz.shape[1]
    sqrdists = _square_distance(new_xyz, xyz)
    group_idx = jnp.broadcast_to(jnp.arange(N, dtype=jnp.int32)[None, None, :], (B, S, N))
    group_idx = jnp.where(sqrdists > radius ** 2, N, group_idx)
    group_idx = jnp.sort(group_idx, axis=-1)[:, :, :nsample]
    group_first = group_idx[:, :, :1]
    group_idx = jnp.where(group_idx == N, group_first, group_idx)
    return group_idx


def _conv_bn_relu(x, W, b, gamma, beta):
    y = jnp.einsum('oc,bcks->boks', W, x) + b[None, :, None, None]
    mean = jnp.mean(y, axis=(0, 2, 3), keepdims=True)
    var = jnp.var(y, axis=(0, 2, 3), keepdims=True)
    y = (y - mean) / jnp.sqrt(var + 1e-5)
    y = gamma[None, :, None, None] * y + beta[None, :, None, None]
    return jax.nn.relu(y)


def _sa_msg(xyz_bcn, points_bcn, cfg, branch_params):
    npoint, radii, nsamples, in_ch, mlps = cfg
    xyz = jnp.transpose(xyz_bcn, (0, 2, 1))
    points = jnp.transpose(points_bcn, (0, 2, 1))
    fps_idx, new_xyz = _fps_pallas(xyz_bcn, npoint)
    outs = []
    for i in range(len(radii)):
        K = nsamples[i]
        gidx = _ballq_pallas(radii[i], K, xyz_bcn, new_xyz)
        grouped_xyz = _index_points(xyz, gidx) - new_xyz[:, :, None, :]
        grouped_points = jnp.concatenate([_index_points(points, gidx), grouped_xyz], axis=-1)
        g = jnp.transpose(grouped_points, (0, 3, 2, 1))
        for (W, b, gamma, beta) in branch_params[i]:
            g = _conv_bn_relu(g, W, b, gamma, beta)
        outs.append(jnp.max(g, axis=2))
    return jnp.transpose(new_xyz, (0, 2, 1)), jnp.concatenate(outs, axis=1)


def _final_kernel(mean_ref, logv_ref, noise_ref, out_ref):
    logv = jnp.clip(logv_ref[...], -30.0, 20.0)
    stdev = jnp.sqrt(jnp.exp(logv))
    out_ref[...] = (mean_ref[...] + stdev * noise_ref[...]) * 0.18215


def kernel(xyz, noise, params):
    l_xyz = xyz[:, :3, :]
    l_points = xyz
    for cfg, bp in zip(_SA_CFGS, params):
        l_xyz, l_points = _sa_msg(l_xyz, l_points, cfg, bp)
    mean, log_variance = jnp.split(l_points, 2, axis=1)
    x = pl.pallas_call(
        _final_kernel,
        out_shape=jax.ShapeDtypeStruct(mean.shape, mean.dtype),
    )(mean, log_variance, noise)
    return x, l_xyz


# speedup vs baseline: 5.4108x; 3.6742x over previous
import functools

import jax
import jax.numpy as jnp
from jax.experimental import pallas as pl
from jax.experimental.pallas import tpu as pltpu
from jax.experimental.pallas import tpu_sc as plsc

_SA_CFGS = [
    (1024, (0.05, 0.1), (16, 32), 9, ((16, 16, 32), (32, 32, 64))),
    (512, (0.1, 0.2), (16, 32), 96, ((64, 64, 128), (64, 96, 128))),
    (256, (0.2, 0.4), (16, 32), 256, ((128, 196, 256), (128, 196, 256))),
    (64, (0.4, 0.8), (16, 32), 512, ((256, 256, 4), (256, 384, 4))),
]


def _pad16(n):
    return ((n + 15) // 16) * 16


def _fps_body(npoint, xyz_ref, idx_ref, cx_ref, cy_ref, cz_ref, dist_ref):
    B = xyz_ref.shape[0]
    N = xyz_ref.shape[2]
    S = npoint
    x = xyz_ref[:, 0, :]
    y = xyz_ref[:, 1, :]
    z = xyz_ref[:, 2, :]
    dist_ref[...] = jnp.full((B, N), 1e10, dtype=jnp.float32)
    lane_n = jax.lax.broadcasted_iota(jnp.int32, (B, N), 1)
    lane_s = jax.lax.broadcasted_iota(jnp.int32, (B, S), 1)

    def step(t, far):
        sel_t = lane_s == t
        idx_ref[...] = jnp.where(sel_t, far, idx_ref[...])
        onehot = lane_n == far
        cx = jnp.sum(jnp.where(onehot, x, 0.0), axis=1, keepdims=True)
        cy = jnp.sum(jnp.where(onehot, y, 0.0), axis=1, keepdims=True)
        cz = jnp.sum(jnp.where(onehot, z, 0.0), axis=1, keepdims=True)
        cx_ref[...] = jnp.where(sel_t, cx, cx_ref[...])
        cy_ref[...] = jnp.where(sel_t, cy, cy_ref[...])
        cz_ref[...] = jnp.where(sel_t, cz, cz_ref[...])
        dx = x - cx
        dy = y - cy
        dz = z - cz
        d = (dx * dx + dy * dy) + dz * dz
        dnew = jnp.minimum(dist_ref[...], d)
        dist_ref[...] = dnew
        maxv = jnp.max(dnew, axis=1, keepdims=True)
        far_next = jnp.min(jnp.where(dnew == maxv, lane_n, N), axis=1,
                           keepdims=True).astype(jnp.int32)
        return far_next

    jax.lax.fori_loop(0, S, step, jnp.zeros((B, 1), jnp.int32))


def _fps_pallas(xyz_bcn, npoint):
    B, _, N = xyz_bcn.shape
    out_shape = [
        jax.ShapeDtypeStruct((B, npoint), jnp.int32),
        jax.ShapeDtypeStruct((B, npoint), jnp.float32),
        jax.ShapeDtypeStruct((B, npoint), jnp.float32),
        jax.ShapeDtypeStruct((B, npoint), jnp.float32),
    ]
    idx, cx, cy, cz = pl.pallas_call(
        functools.partial(_fps_body, npoint),
        out_shape=out_shape,
        scratch_shapes=[pltpu.VMEM((B, N), jnp.float32)],
    )(xyz_bcn)
    new_xyz = jnp.stack([cx, cy, cz], axis=-1)
    return idx, new_xyz


def _ballq_body(r2, K, N, Sblk, xyz_ref, nx_ref, gidx_ref):
    b = pl.program_id(0)
    si = pl.program_id(1)
    x = xyz_ref[b, 0:1, :]
    y = xyz_ref[b, 1:2, :]
    z = xyz_ref[b, 2:3, :]
    c = nx_ref[b, pl.ds(si * Sblk, Sblk), :]
    cx = c[:, 0:1]
    cy = c[:, 1:2]
    cz = c[:, 2:3]
    xmat = jnp.concatenate([x, y, z], axis=0).astype(jnp.bfloat16)
    dot = jax.lax.dot_general(
        c.astype(jnp.bfloat16), xmat, (((1,), (0,)), ((), ())),
        preferred_element_type=jnp.float32)
    csq = (cx * cx + cy * cy) + cz * cz
    xsq = (x * x + y * y) + z * z
    dist = (-2.0 * dot + csq) + xsq
    jf = jax.lax.broadcasted_iota(jnp.int32, (Sblk, N), 1).astype(jnp.float32)
    fN = jnp.float32(N)
    d0 = jnp.where(dist <= r2, jf, fN)
    cols = []
    for _ in range(K):
        m = jnp.min(d0, axis=1, keepdims=True)
        cols.append(m)
        d0 = jnp.where(d0 == m, fN, d0)
    out = jnp.concatenate(cols, axis=1)
    out = jnp.where(out == fN, cols[0], out)
    out = jnp.minimum(out, fN - 1.0)
    gidx_ref[b, pl.ds(si * Sblk, Sblk), :] = out.astype(jnp.int32) + b * N


def _ballq_pallas(radius, K, xyz_b3n, new_xyz):
    B, _, N = xyz_b3n.shape
    S = new_xyz.shape[1]
    Sblk = 8
    grid = (B, S // Sblk)
    return pl.pallas_call(
        functools.partial(_ballq_body, radius * radius, K, N, Sblk),
        grid=grid,
        in_specs=[
            pl.BlockSpec((B, 3, N), lambda b, s: (0, 0, 0)),
            pl.BlockSpec((B, S, 3), lambda b, s: (0, 0, 0)),
        ],
        out_specs=pl.BlockSpec((B, S, K), lambda b, s: (0, 0, 0)),
        out_shape=jax.ShapeDtypeStruct((B, S, K), jnp.int32),
    )(xyz_b3n, new_xyz)


def _sc_gather(Fchunks, idx1):
    if len(Fchunks) > 3:
        return jnp.concatenate(
            [_sc_gather(Fchunks[:3], idx1), _sc_gather(Fchunks[3:], idx1)],
            axis=-1)
    R = idx1.shape[0]
    nch = len(Fchunks)
    Ptot = 128 * nch
    W = 128
    i1 = idx1.reshape(1, R)
    mesh = plsc.VectorSubcoreMesh(core_axis_name="c", subcore_axis_name="s")

    @pl.kernel(out_type=jax.ShapeDtypeStruct((R, Ptot), jnp.float32),
               mesh=mesh)
    def gather_kernel(*refs):
        f_hbms = refs[:nch]
        i1_hbm = refs[nch]
        o_hbm = refs[nch + 1]

        def body(i1_v, o_v):
            for j in range(nch):
                pltpu.sync_copy(f_hbms[j].at[i1_v.at[0]],
                                o_v.at[:, pl.ds(128 * j, 128)])

        pltpu.emit_pipeline(
            body,
            grid=(R // W,),
            in_specs=[pl.BlockSpec((1, W), lambda i: (0, i))],
            out_specs=[pl.BlockSpec((W, Ptot), lambda i: (i, 0))],
            core_axis_name=("c", "s"),
            dimension_semantics=(pltpu.PARALLEL,),
        )(i1_hbm, o_hbm)

    return gather_kernel(*Fchunks, i1)


def _conv1_body(C, K, w_ref, pp_ref, nx_ref, g_ref, y_ref, st_ref):
    g = g_ref[...]
    Rblk, Ptot = g.shape
    ctr = nx_ref[...]
    Cb = ctr.shape[0]
    rr = jax.lax.broadcasted_iota(jnp.int32, (Rblk, Cb), 0) // K
    cc = jax.lax.broadcasted_iota(jnp.int32, (Rblk, Cb), 1)
    E = (rr == cc).astype(jnp.float32)
    crep = jax.lax.dot_general(E, ctr, (((1,), (0,)), ((), ())),
                               preferred_element_type=jnp.float32,
                               precision=jax.lax.Precision.HIGHEST)
    lanepos = C % 128
    chunk = C // 128
    rolled = pltpu.roll(crep, lanepos, axis=1)
    lane = jax.lax.broadcasted_iota(jnp.int32, (Rblk, 128), 1)
    masked = jnp.where((lane >= lanepos) & (lane < lanepos + 3), rolled, 0.0)
    nch = Ptot // 128
    parts = ([jnp.zeros((Rblk, 128 * chunk), jnp.float32)] if chunk else []) \
        + [masked] \
        + ([jnp.zeros((Rblk, 128 * (nch - chunk - 1)), jnp.float32)]
           if nch - chunk - 1 else [])
    corr = jnp.concatenate(parts, axis=1) if len(parts) > 1 else parts[0]
    xin = (g - corr).astype(jnp.bfloat16)
    y = jax.lax.dot_general(xin, w_ref[...], (((1,), (0,)), ((), ())),
                            preferred_element_type=jnp.float32)
    y = y + pp_ref[0:1, :]
    y_ref[...] = y
    ps = jnp.sum(y, axis=0, keepdims=True)
    pq = jnp.sum(y * y, axis=0, keepdims=True)

    @pl.when(pl.program_id(0) == 0)
    def _():
        st_ref[...] = jnp.zeros_like(st_ref)

    st_ref[0:1, :] += ps
    st_ref[1:2, :] += pq


def _norm_conv_body(n, w_ref, ppp_ref, stp_ref, pp_ref, yp_ref, y_ref, st_ref):
    mean = stp_ref[0:1, :] * (1.0 / n)
    var = stp_ref[1:2, :] * (1.0 / n) - mean * mean
    s = ppp_ref[1:2, :] / jnp.sqrt(var + 1e-5)
    t = ppp_ref[2:3, :] - mean * s
    a = jnp.maximum(yp_ref[...] * s + t, 0.0).astype(jnp.bfloat16)
    y = jax.lax.dot_general(a, w_ref[...], (((1,), (0,)), ((), ())),
                            preferred_element_type=jnp.float32)
    y = y + pp_ref[0:1, :]
    y_ref[...] = y
    ps = jnp.sum(y, axis=0, keepdims=True)
    pq = jnp.sum(y * y, axis=0, keepdims=True)

    @pl.when(pl.program_id(0) == 0)
    def _():
        st_ref[...] = jnp.zeros_like(st_ref)

    st_ref[0:1, :] += ps
    st_ref[1:2, :] += pq


def _norm_max_body(n, ppp_ref, stp_ref, y_ref, o_ref):
    mean = stp_ref[0:1, :] * (1.0 / n)
    var = stp_ref[1:2, :] * (1.0 / n) - mean * mean
    s = (ppp_ref[1:2, :] / jnp.sqrt(var + 1e-5))[None]
    t = (ppp_ref[2:3, :] - stp_ref[0:1, :] * (1.0 / n) * s[0])[None]
    a = jnp.maximum(y_ref[...] * s + t, 0.0)
    o_ref[...] = jnp.max(a, axis=1)


def _conv_branch(G, nx_pad, C, Ws, PPs, B, S, K):
    R = G.shape[0]
    Ptot = G.shape[1]
    Rblk = 2048 if Ptot <= 128 else 1024
    grid = (R // Rblk,)
    Cb = Rblk // K

    def call_conv(body, extra_in, yprev, Cout):
        return pl.pallas_call(
            body,
            grid=grid,
            in_specs=[pl.BlockSpec(a.shape, lambda i, nd=a.ndim: (0,) * nd)
                      for a in extra_in]
            + [pl.BlockSpec((Rblk, yprev.shape[1]), lambda i: (i, 0))],
            out_specs=[
                pl.BlockSpec((Rblk, Cout), lambda i: (i, 0)),
                pl.BlockSpec((8, Cout), lambda i: (0, 0)),
            ],
            out_shape=[
                jax.ShapeDtypeStruct((R, Cout), jnp.float32),
                jax.ShapeDtypeStruct((8, Cout), jnp.float32),
            ],
        )(*extra_in, yprev)

    W1, W2, W3 = Ws
    PP1, PP2, PP3 = PPs
    C1 = W1.shape[1]
    y1, st1 = pl.pallas_call(
        functools.partial(_conv1_body, C, K),
        grid=grid,
        in_specs=[
            pl.BlockSpec(W1.shape, lambda i: (0, 0)),
            pl.BlockSpec(PP1.shape, lambda i: (0, 0)),
            pl.BlockSpec((Cb, 128), lambda i: (i, 0)),
            pl.BlockSpec((Rblk, Ptot), lambda i: (i, 0)),
        ],
        out_specs=[
            pl.BlockSpec((Rblk, C1), lambda i: (i, 0)),
            pl.BlockSpec((8, C1), lambda i: (0, 0)),
        ],
        out_shape=[
            jax.ShapeDtypeStruct((R, C1), jnp.float32),
            jax.ShapeDtypeStruct((8, C1), jnp.float32),
        ],
    )(W1, PP1, nx_pad, G)
    y2, st2 = call_conv(functools.partial(_norm_conv_body, float(R)),
                        [W2, PP1, st1, PP2], y1, W2.shape[1])
    y3, st3 = call_conv(functools.partial(_norm_conv_body, float(R)),
                        [W3, PP2, st2, PP3], y2, W3.shape[1])
    C3 = W3.shape[1]
    y3r = y3.reshape(B * S, K, C3)
    Mblk = 64
    out = pl.pallas_call(
        functools.partial(_norm_max_body, float(R)),
        grid=(B * S // Mblk,),
        in_specs=[
            pl.BlockSpec(PP3.shape, lambda i: (0, 0)),
            pl.BlockSpec(st3.shape, lambda i: (0, 0)),
            pl.BlockSpec((Mblk, K, C3), lambda i: (i, 0, 0)),
        ],
        out_specs=pl.BlockSpec((Mblk, C3), lambda i: (i, 0)),
        out_shape=jax.ShapeDtypeStruct((B * S, C3), jnp.float32),
    )(PP3, st3, y3r)
    return out


def _final_kernel(mean_ref, logv_ref, noise_ref, out_ref):
    logv = jnp.clip(logv_ref[...], -30.0, 20.0)
    stdev = jnp.sqrt(jnp.exp(logv))
    out_ref[...] = (mean_ref[...] + stdev * noise_ref[...]) * 0.18215


def _prep_params(params):
    prepped = []
    for li, ((_, _, _, C, mlps), branches) in enumerate(zip(_SA_CFGS, params)):
        P1c = 128 * (-(-(C + 3) // 128))
        pb = []
        for bi, layers in enumerate(branches):
            Ws, PPs = [], []
            for lj, (W, b, gamma, beta) in enumerate(layers):
                Cout, Cin = W.shape
                Cop = _pad16(Cout) if Cout < 16 else Cout
                if lj == 0:
                    Wt = jnp.zeros((P1c, Cop), jnp.float32)
                    Wt = Wt.at[:C + 3, :Cout].set(W.T)
                else:
                    Wt = jnp.zeros((Cin, Cop), jnp.float32)
                    Wt = Wt.at[:, :Cout].set(W.T)
                PP = jnp.zeros((8, Cop), jnp.float32)
                PP = PP.at[0, :Cout].set(b)
                PP = PP.at[1, :Cout].set(gamma)
                PP = PP.at[2, :Cout].set(beta)
                Ws.append(Wt.astype(jnp.bfloat16))
                PPs.append(PP)
            pb.append((Ws, PPs))
        prepped.append(pb)
    return prepped


def kernel(xyz, noise, params):
    B = xyz.shape[0]
    prepped = _prep_params(params)
    l_xyz_b3n = xyz[:, :3, :]
    feats = jnp.transpose(xyz, (0, 2, 1))
    new_xyz = None
    outs = None
    for (cfg, pb) in zip(_SA_CFGS, prepped):
        S, radii, Ks, C, _ = cfg
        N = feats.shape[1]
        xyz_bn3 = jnp.transpose(l_xyz_b3n, (0, 2, 1))
        P1c = 128 * (-(-(C + 3) // 128))
        F = jnp.concatenate(
            [feats, xyz_bn3,
             jnp.zeros((B, N, P1c - (C + 3)), jnp.float32)], axis=-1
        ).reshape(B * N, P1c)
        Fchunks = [F[:, 128 * j:128 * (j + 1)] for j in range(P1c // 128)]
        fps_idx, new_xyz = _fps_pallas(l_xyz_b3n, S)
        nx_pad = jnp.concatenate(
            [new_xyz, jnp.zeros((B, S, 125), jnp.float32)], axis=-1
        ).reshape(B * S, 128)
        outs = []
        for bi in range(2):
            K = Ks[bi]
            gflat = _ballq_pallas(radii[bi], K, l_xyz_b3n, new_xyz)
            idx1 = gflat.reshape(B * S * K)
            G = _sc_gather(Fchunks, idx1)
            Ws, PPs = pb[bi]
            outs.append(_conv_branch(G, nx_pad, C, Ws, PPs, B, S, K))
        feats = jnp.concatenate(
            [o.reshape(B, S, o.shape[1]) for o in outs], axis=-1)
        l_xyz_b3n = jnp.transpose(new_xyz, (0, 2, 1))

    S = _SA_CFGS[-1][0]
    mean = jnp.transpose(outs[0].reshape(B, S, -1)[:, :, :4], (0, 2, 1))
    logv = jnp.transpose(outs[1].reshape(B, S, -1)[:, :, :4], (0, 2, 1))
    x = pl.pallas_call(
        _final_kernel,
        out_shape=jax.ShapeDtypeStruct(mean.shape, mean.dtype),
    )(mean, logv, noise)
    return x, l_xyz_b3n


# speedup vs baseline: 7.0671x; 1.3061x over previous
import functools

import jax
import jax.numpy as jnp
from jax.experimental import pallas as pl
from jax.experimental.pallas import tpu as pltpu
from jax.experimental.pallas import tpu_sc as plsc

_SA_CFGS = [
    (1024, (0.05, 0.1), (16, 32), 9, ((16, 16, 32), (32, 32, 64))),
    (512, (0.1, 0.2), (16, 32), 96, ((64, 64, 128), (64, 96, 128))),
    (256, (0.2, 0.4), (16, 32), 256, ((128, 196, 256), (128, 196, 256))),
    (64, (0.4, 0.8), (16, 32), 512, ((256, 256, 4), (256, 384, 4))),
]


def _pad16(n):
    return ((n + 15) // 16) * 16


def _fps_body(npoint, xyz_ref, idx_ref, cx_ref, cy_ref, cz_ref, dist_ref):
    B = xyz_ref.shape[0]
    N = xyz_ref.shape[2]
    S = npoint
    x = xyz_ref[:, 0, :]
    y = xyz_ref[:, 1, :]
    z = xyz_ref[:, 2, :]
    dist_ref[...] = jnp.full((B, N), 1e10, dtype=jnp.float32)
    lane_n = jax.lax.broadcasted_iota(jnp.int32, (B, N), 1)
    lane_s = jax.lax.broadcasted_iota(jnp.int32, (B, S), 1)

    def step(t, far):
        sel_t = lane_s == t
        idx_ref[...] = jnp.where(sel_t, far, idx_ref[...])
        onehot = lane_n == far
        cx = jnp.sum(jnp.where(onehot, x, 0.0), axis=1, keepdims=True)
        cy = jnp.sum(jnp.where(onehot, y, 0.0), axis=1, keepdims=True)
        cz = jnp.sum(jnp.where(onehot, z, 0.0), axis=1, keepdims=True)
        cx_ref[...] = jnp.where(sel_t, cx, cx_ref[...])
        cy_ref[...] = jnp.where(sel_t, cy, cy_ref[...])
        cz_ref[...] = jnp.where(sel_t, cz, cz_ref[...])
        dx = x - cx
        dy = y - cy
        dz = z - cz
        d = (dx * dx + dy * dy) + dz * dz
        dnew = jnp.minimum(dist_ref[...], d)
        dist_ref[...] = dnew
        maxv = jnp.max(dnew, axis=1, keepdims=True)
        far_next = jnp.min(jnp.where(dnew == maxv, lane_n, N), axis=1,
                           keepdims=True).astype(jnp.int32)
        return far_next

    jax.lax.fori_loop(0, S, step, jnp.zeros((B, 1), jnp.int32))


def _fps_pallas(xyz_bcn, npoint):
    B, _, N = xyz_bcn.shape
    out_shape = [
        jax.ShapeDtypeStruct((B, npoint), jnp.int32),
        jax.ShapeDtypeStruct((B, npoint), jnp.float32),
        jax.ShapeDtypeStruct((B, npoint), jnp.float32),
        jax.ShapeDtypeStruct((B, npoint), jnp.float32),
    ]
    idx, cx, cy, cz = pl.pallas_call(
        functools.partial(_fps_body, npoint),
        out_shape=out_shape,
        scratch_shapes=[pltpu.VMEM((B, N), jnp.float32)],
    )(xyz_bcn)
    new_xyz = jnp.stack([cx, cy, cz], axis=-1)
    return idx, new_xyz


def _ballq_body(r2a, Ka, r2b, Kb, N, Sblk, xyz_ref, nx_ref, ga_ref, gb_ref):
    b = pl.program_id(0)
    si = pl.program_id(1)
    x = xyz_ref[b, 0:1, :]
    y = xyz_ref[b, 1:2, :]
    z = xyz_ref[b, 2:3, :]
    c = nx_ref[b, pl.ds(si * Sblk, Sblk), :]
    cx = c[:, 0:1]
    cy = c[:, 1:2]
    cz = c[:, 2:3]
    xmat = jnp.concatenate([x, y, z], axis=0).astype(jnp.bfloat16)
    dot = jax.lax.dot_general(
        c.astype(jnp.bfloat16), xmat, (((1,), (0,)), ((), ())),
        preferred_element_type=jnp.float32)
    csq = (cx * cx + cy * cy) + cz * cz
    xsq = (x * x + y * y) + z * z
    dist = (-2.0 * dot + csq) + xsq
    jf = jax.lax.broadcasted_iota(jnp.int32, (Sblk, N), 1).astype(jnp.float32)
    fN = jnp.float32(N)
    da = jnp.where(dist <= r2a, jf, fN)
    db = jnp.where(dist <= r2b, jf, fN)
    cols_a, cols_b = [], []
    for k in range(max(Ka, Kb)):
        if k < Ka:
            ma = jnp.min(da, axis=1, keepdims=True)
            cols_a.append(ma)
            da = jnp.where(da == ma, fN, da)
        if k < Kb:
            mb = jnp.min(db, axis=1, keepdims=True)
            cols_b.append(mb)
            db = jnp.where(db == mb, fN, db)
    for cols, K, ref in ((cols_a, Ka, ga_ref), (cols_b, Kb, gb_ref)):
        out = jnp.concatenate(cols, axis=1)
        out = jnp.where(out == fN, cols[0], out)
        out = jnp.minimum(out, fN - 1.0)
        ref[b, pl.ds(si * Sblk, Sblk), :] = out.astype(jnp.int32) + b * N


def _ballq_pallas(radii, Ks, xyz_b3n, new_xyz):
    B, _, N = xyz_b3n.shape
    S = new_xyz.shape[1]
    Sblk = 8
    grid = (B, S // Sblk)
    Ka, Kb = Ks
    return pl.pallas_call(
        functools.partial(_ballq_body, radii[0] ** 2, Ka, radii[1] ** 2, Kb,
                          N, Sblk),
        grid=grid,
        in_specs=[
            pl.BlockSpec((B, 3, N), lambda b, s: (0, 0, 0)),
            pl.BlockSpec((B, S, 3), lambda b, s: (0, 0, 0)),
        ],
        out_specs=[
            pl.BlockSpec((B, S, Ka), lambda b, s: (0, 0, 0)),
            pl.BlockSpec((B, S, Kb), lambda b, s: (0, 0, 0)),
        ],
        out_shape=[
            jax.ShapeDtypeStruct((B, S, Ka), jnp.int32),
            jax.ShapeDtypeStruct((B, S, Kb), jnp.int32),
        ],
    )(xyz_b3n, new_xyz)


def _sc_gather(Fchunks, idx1):
    if len(Fchunks) > 3:
        return jnp.concatenate(
            [_sc_gather(Fchunks[:3], idx1), _sc_gather(Fchunks[3:], idx1)],
            axis=-1)
    R = idx1.shape[0]
    nch = len(Fchunks)
    Ptot = 128 * nch
    W = 256 if nch == 1 else 128
    i1 = idx1.reshape(1, R)
    mesh = plsc.VectorSubcoreMesh(core_axis_name="c", subcore_axis_name="s")

    @pl.kernel(out_type=jax.ShapeDtypeStruct((R, Ptot), jnp.float32),
               mesh=mesh)
    def gather_kernel(*refs):
        f_hbms = refs[:nch]
        i1_hbm = refs[nch]
        o_hbm = refs[nch + 1]

        def body(i1_v, o_v):
            for j in range(nch):
                pltpu.sync_copy(f_hbms[j].at[i1_v.at[0]],
                                o_v.at[:, pl.ds(128 * j, 128)])

        pltpu.emit_pipeline(
            body,
            grid=(R // W,),
            in_specs=[pl.BlockSpec((1, W), lambda i: (0, i))],
            out_specs=[pl.BlockSpec((W, Ptot), lambda i: (i, 0))],
            core_axis_name=("c", "s"),
            dimension_semantics=(pltpu.PARALLEL,),
        )(i1_hbm, o_hbm)

    return gather_kernel(*Fchunks, i1)


def _conv1_body(C, K, w_ref, pp_ref, nx_ref, g_ref, y_ref, st_ref):
    g = g_ref[...]
    Rblk, Ptot = g.shape
    ctr = nx_ref[...]
    Cb = ctr.shape[0]
    rr = jax.lax.broadcasted_iota(jnp.int32, (Rblk, Cb), 0) // K
    cc = jax.lax.broadcasted_iota(jnp.int32, (Rblk, Cb), 1)
    E = (rr == cc).astype(jnp.float32)
    crep = jax.lax.dot_general(E, ctr, (((1,), (0,)), ((), ())),
                               preferred_element_type=jnp.float32,
                               precision=jax.lax.Precision.HIGHEST)
    lanepos = C % 128
    chunk = C // 128
    rolled = pltpu.roll(crep, lanepos, axis=1)
    lane = jax.lax.broadcasted_iota(jnp.int32, (Rblk, 128), 1)
    masked = jnp.where((lane >= lanepos) & (lane < lanepos + 3), rolled, 0.0)
    nch = Ptot // 128
    parts = ([jnp.zeros((Rblk, 128 * chunk), jnp.float32)] if chunk else []) \
        + [masked] \
        + ([jnp.zeros((Rblk, 128 * (nch - chunk - 1)), jnp.float32)]
           if nch - chunk - 1 else [])
    corr = jnp.concatenate(parts, axis=1) if len(parts) > 1 else parts[0]
    xin = (g - corr).astype(jnp.bfloat16)
    y = jax.lax.dot_general(xin, w_ref[...], (((1,), (0,)), ((), ())),
                            preferred_element_type=jnp.float32)
    y = y + pp_ref[0:1, :]
    y_ref[...] = y
    ps = jnp.sum(y, axis=0, keepdims=True)
    pq = jnp.sum(y * y, axis=0, keepdims=True)

    @pl.when(pl.program_id(0) == 0)
    def _():
        st_ref[...] = jnp.zeros_like(st_ref)

    st_ref[0:1, :] += ps
    st_ref[1:2, :] += pq


def _norm_conv_body(n, w_ref, ppp_ref, stp_ref, pp_ref, yp_ref, y_ref, st_ref):
    mean = stp_ref[0:1, :] * (1.0 / n)
    var = stp_ref[1:2, :] * (1.0 / n) - mean * mean
    s = ppp_ref[1:2, :] / jnp.sqrt(var + 1e-5)
    t = ppp_ref[2:3, :] - mean * s
    a = jnp.maximum(yp_ref[...] * s + t, 0.0).astype(jnp.bfloat16)
    y = jax.lax.dot_general(a, w_ref[...], (((1,), (0,)), ((), ())),
                            preferred_element_type=jnp.float32)
    y = y + pp_ref[0:1, :]
    y_ref[...] = y
    ps = jnp.sum(y, axis=0, keepdims=True)
    pq = jnp.sum(y * y, axis=0, keepdims=True)

    @pl.when(pl.program_id(0) == 0)
    def _():
        st_ref[...] = jnp.zeros_like(st_ref)

    st_ref[0:1, :] += ps
    st_ref[1:2, :] += pq


def _norm_max_body(n, ppp_ref, stp_ref, y_ref, o_ref):
    mean = stp_ref[0:1, :] * (1.0 / n)
    var = stp_ref[1:2, :] * (1.0 / n) - mean * mean
    s = (ppp_ref[1:2, :] / jnp.sqrt(var + 1e-5))[None]
    t = (ppp_ref[2:3, :] - stp_ref[0:1, :] * (1.0 / n) * s[0])[None]
    a = jnp.maximum(y_ref[...] * s + t, 0.0)
    o_ref[...] = jnp.max(a, axis=1)


def _conv_branch(G, nx_pad, C, Ws, PPs, B, S, K):
    R = G.shape[0]
    Ptot = G.shape[1]
    Rblk = 2048 if Ptot <= 128 else 1024
    grid = (R // Rblk,)
    Cb = Rblk // K

    def call_conv(body, extra_in, yprev, Cout):
        return pl.pallas_call(
            body,
            grid=grid,
            in_specs=[pl.BlockSpec(a.shape, lambda i, nd=a.ndim: (0,) * nd)
                      for a in extra_in]
            + [pl.BlockSpec((Rblk, yprev.shape[1]), lambda i: (i, 0))],
            out_specs=[
                pl.BlockSpec((Rblk, Cout), lambda i: (i, 0)),
                pl.BlockSpec((8, Cout), lambda i: (0, 0)),
            ],
            out_shape=[
                jax.ShapeDtypeStruct((R, Cout), jnp.float32),
                jax.ShapeDtypeStruct((8, Cout), jnp.float32),
            ],
        )(*extra_in, yprev)

    W1, W2, W3 = Ws
    PP1, PP2, PP3 = PPs
    C1 = W1.shape[1]
    y1, st1 = pl.pallas_call(
        functools.partial(_conv1_body, C, K),
        grid=grid,
        in_specs=[
            pl.BlockSpec(W1.shape, lambda i: (0, 0)),
            pl.BlockSpec(PP1.shape, lambda i: (0, 0)),
            pl.BlockSpec((Cb, 128), lambda i: (i, 0)),
            pl.BlockSpec((Rblk, Ptot), lambda i: (i, 0)),
        ],
        out_specs=[
            pl.BlockSpec((Rblk, C1), lambda i: (i, 0)),
            pl.BlockSpec((8, C1), lambda i: (0, 0)),
        ],
        out_shape=[
            jax.ShapeDtypeStruct((R, C1), jnp.float32),
            jax.ShapeDtypeStruct((8, C1), jnp.float32),
        ],
    )(W1, PP1, nx_pad, G)
    y2, st2 = call_conv(functools.partial(_norm_conv_body, float(R)),
                        [W2, PP1, st1, PP2], y1, W2.shape[1])
    y3, st3 = call_conv(functools.partial(_norm_conv_body, float(R)),
                        [W3, PP2, st2, PP3], y2, W3.shape[1])
    C3 = W3.shape[1]
    y3r = y3.reshape(B * S, K, C3)
    Mblk = 64
    out = pl.pallas_call(
        functools.partial(_norm_max_body, float(R)),
        grid=(B * S // Mblk,),
        in_specs=[
            pl.BlockSpec(PP3.shape, lambda i: (0, 0)),
            pl.BlockSpec(st3.shape, lambda i: (0, 0)),
            pl.BlockSpec((Mblk, K, C3), lambda i: (i, 0, 0)),
        ],
        out_specs=pl.BlockSpec((Mblk, C3), lambda i: (i, 0)),
        out_shape=jax.ShapeDtypeStruct((B * S, C3), jnp.float32),
    )(PP3, st3, y3r)
    return out


def _final_kernel(mean_ref, logv_ref, noise_ref, out_ref):
    logv = jnp.clip(logv_ref[...], -30.0, 20.0)
    stdev = jnp.sqrt(jnp.exp(logv))
    out_ref[...] = (mean_ref[...] + stdev * noise_ref[...]) * 0.18215


def _prep_params(params):
    prepped = []
    for li, ((_, _, _, C, mlps), branches) in enumerate(zip(_SA_CFGS, params)):
        P1c = 128 * (-(-(C + 3) // 128))
        pb = []
        for bi, layers in enumerate(branches):
            Ws, PPs = [], []
            for lj, (W, b, gamma, beta) in enumerate(layers):
                Cout, Cin = W.shape
                Cop = _pad16(Cout) if Cout < 16 else Cout
                if lj == 0:
                    Wt = jnp.zeros((P1c, Cop), jnp.float32)
                    Wt = Wt.at[:C + 3, :Cout].set(W.T)
                else:
                    Wt = jnp.zeros((Cin, Cop), jnp.float32)
                    Wt = Wt.at[:, :Cout].set(W.T)
                PP = jnp.zeros((8, Cop), jnp.float32)
                PP = PP.at[0, :Cout].set(b)
                PP = PP.at[1, :Cout].set(gamma)
                PP = PP.at[2, :Cout].set(beta)
                Ws.append(Wt.astype(jnp.bfloat16))
                PPs.append(PP)
            pb.append((Ws, PPs))
        prepped.append(pb)
    return prepped


def kernel(xyz, noise, params):
    B = xyz.shape[0]
    prepped = _prep_params(params)
    l_xyz_b3n = xyz[:, :3, :]
    feats = jnp.transpose(xyz, (0, 2, 1))
    new_xyz = None
    outs = None
    for (cfg, pb) in zip(_SA_CFGS, prepped):
        S, radii, Ks, C, _ = cfg
        N = feats.shape[1]
        xyz_bn3 = jnp.transpose(l_xyz_b3n, (0, 2, 1))
        P1c = 128 * (-(-(C + 3) // 128))
        F = jnp.concatenate(
            [feats, xyz_bn3,
             jnp.zeros((B, N, P1c - (C + 3)), jnp.float32)], axis=-1
        ).reshape(B * N, P1c)
        Fchunks = [F[:, 128 * j:128 * (j + 1)] for j in range(P1c // 128)]
        fps_idx, new_xyz = _fps_pallas(l_xyz_b3n, S)
        nx_pad = jnp.concatenate(
            [new_xyz, jnp.zeros((B, S, 125), jnp.float32)], axis=-1
        ).reshape(B * S, 128)
        gflats = _ballq_pallas(radii, Ks, l_xyz_b3n, new_xyz)
        outs = []
        for bi in range(2):
            K = Ks[bi]
            idx1 = gflats[bi].reshape(B * S * K)
            G = _sc_gather(Fchunks, idx1)
            Ws, PPs = pb[bi]
            outs.append(_conv_branch(G, nx_pad, C, Ws, PPs, B, S, K))
        feats = jnp.concatenate(
            [o.reshape(B, S, o.shape[1]) for o in outs], axis=-1)
        l_xyz_b3n = jnp.transpose(new_xyz, (0, 2, 1))

    S = _SA_CFGS[-1][0]
    mean = jnp.transpose(outs[0].reshape(B, S, -1)[:, :, :4], (0, 2, 1))
    logv = jnp.transpose(outs[1].reshape(B, S, -1)[:, :, :4], (0, 2, 1))
    x = pl.pallas_call(
        _final_kernel,
        out_shape=jax.ShapeDtypeStruct(mean.shape, mean.dtype),
    )(mean, logv, noise)
    return x, l_xyz_b3n


# speedup vs baseline: 9.8242x; 1.3901x over previous
import functools

import jax
import jax.numpy as jnp
from jax.experimental import pallas as pl
from jax.experimental.pallas import tpu as pltpu
from jax.experimental.pallas import tpu_sc as plsc

_SA_CFGS = [
    (1024, (0.05, 0.1), (16, 32), 9, ((16, 16, 32), (32, 32, 64))),
    (512, (0.1, 0.2), (16, 32), 96, ((64, 64, 128), (64, 96, 128))),
    (256, (0.2, 0.4), (16, 32), 256, ((128, 196, 256), (128, 196, 256))),
    (64, (0.4, 0.8), (16, 32), 512, ((256, 256, 4), (256, 384, 4))),
]


def _pad16(n):
    return ((n + 15) // 16) * 16


def _fps_body(npoint, xyz_ref, idx_ref, cx_ref, cy_ref, cz_ref, dist_ref):
    B = xyz_ref.shape[0]
    N = xyz_ref.shape[2]
    S = npoint
    x = xyz_ref[:, 0, :]
    y = xyz_ref[:, 1, :]
    z = xyz_ref[:, 2, :]
    dist_ref[...] = jnp.full((B, N), 1e10, dtype=jnp.float32)
    lane_n = jax.lax.broadcasted_iota(jnp.int32, (B, N), 1)
    lane_s = jax.lax.broadcasted_iota(jnp.int32, (B, S), 1)

    def step(t, far):
        sel_t = lane_s == t
        idx_ref[...] = jnp.where(sel_t, far, idx_ref[...])
        onehot = lane_n == far
        cx = jnp.sum(jnp.where(onehot, x, 0.0), axis=1, keepdims=True)
        cy = jnp.sum(jnp.where(onehot, y, 0.0), axis=1, keepdims=True)
        cz = jnp.sum(jnp.where(onehot, z, 0.0), axis=1, keepdims=True)
        cx_ref[...] = jnp.where(sel_t, cx, cx_ref[...])
        cy_ref[...] = jnp.where(sel_t, cy, cy_ref[...])
        cz_ref[...] = jnp.where(sel_t, cz, cz_ref[...])
        dx = x - cx
        dy = y - cy
        dz = z - cz
        d = (dx * dx + dy * dy) + dz * dz
        dnew = jnp.minimum(dist_ref[...], d)
        dist_ref[...] = dnew
        maxv = jnp.max(dnew, axis=1, keepdims=True)
        far_next = jnp.min(jnp.where(dnew == maxv, lane_n, N), axis=1,
                           keepdims=True).astype(jnp.int32)
        return far_next

    jax.lax.fori_loop(0, S, step, jnp.zeros((B, 1), jnp.int32))


def _fps_pallas(xyz_bcn, npoint):
    B, _, N = xyz_bcn.shape
    out_shape = [
        jax.ShapeDtypeStruct((B, npoint), jnp.int32),
        jax.ShapeDtypeStruct((B, npoint), jnp.float32),
        jax.ShapeDtypeStruct((B, npoint), jnp.float32),
        jax.ShapeDtypeStruct((B, npoint), jnp.float32),
    ]
    idx, cx, cy, cz = pl.pallas_call(
        functools.partial(_fps_body, npoint),
        out_shape=out_shape,
        scratch_shapes=[pltpu.VMEM((B, N), jnp.float32)],
    )(xyz_bcn)
    new_xyz = jnp.stack([cx, cy, cz], axis=-1)
    return idx, new_xyz


def _ballq_body(r2a, Ka, r2b, Kb, N, Sblk, xyz_ref, nx_ref, ga_ref, gb_ref):
    b = pl.program_id(0)
    si = pl.program_id(1)
    x = xyz_ref[b, 0:1, :]
    y = xyz_ref[b, 1:2, :]
    z = xyz_ref[b, 2:3, :]
    c = nx_ref[b, pl.ds(si * Sblk, Sblk), :]
    cx = c[:, 0:1]
    cy = c[:, 1:2]
    cz = c[:, 2:3]
    xmat = jnp.concatenate([x, y, z], axis=0).astype(jnp.bfloat16)
    dot = jax.lax.dot_general(
        c.astype(jnp.bfloat16), xmat, (((1,), (0,)), ((), ())),
        preferred_element_type=jnp.float32)
    csq = (cx * cx + cy * cy) + cz * cz
    xsq = (x * x + y * y) + z * z
    dist = (-2.0 * dot + csq) + xsq
    jf = jax.lax.broadcasted_iota(jnp.int32, (Sblk, N), 1).astype(jnp.float32)
    fN = jnp.float32(N)
    da = jnp.where(dist <= r2a, jf, fN)
    db = jnp.where(dist <= r2b, jf, fN)
    cols_a, cols_b = [], []
    for k in range(max(Ka, Kb)):
        if k < Ka:
            ma = jnp.min(da, axis=1, keepdims=True)
            cols_a.append(ma)
            da = jnp.where(da == ma, fN, da)
        if k < Kb:
            mb = jnp.min(db, axis=1, keepdims=True)
            cols_b.append(mb)
            db = jnp.where(db == mb, fN, db)
    for cols, K, ref in ((cols_a, Ka, ga_ref), (cols_b, Kb, gb_ref)):
        out = jnp.concatenate(cols, axis=1)
        out = jnp.where(out == fN, cols[0], out)
        out = jnp.minimum(out, fN - 1.0)
        ref[b, pl.ds(si * Sblk, Sblk), :] = out.astype(jnp.int32) + b * N


def _ballq_pallas(radii, Ks, xyz_b3n, new_xyz):
    B, _, N = xyz_b3n.shape
    S = new_xyz.shape[1]
    Sblk = 16
    grid = (B, S // Sblk)
    Ka, Kb = Ks
    return pl.pallas_call(
        functools.partial(_ballq_body, radii[0] ** 2, Ka, radii[1] ** 2, Kb,
                          N, Sblk),
        grid=grid,
        in_specs=[
            pl.BlockSpec((B, 3, N), lambda b, s: (0, 0, 0)),
            pl.BlockSpec((B, S, 3), lambda b, s: (0, 0, 0)),
        ],
        out_specs=[
            pl.BlockSpec((B, S, Ka), lambda b, s: (0, 0, 0)),
            pl.BlockSpec((B, S, Kb), lambda b, s: (0, 0, 0)),
        ],
        out_shape=[
            jax.ShapeDtypeStruct((B, S, Ka), jnp.int32),
            jax.ShapeDtypeStruct((B, S, Kb), jnp.int32),
        ],
    )(xyz_b3n, new_xyz)


def _sc_gather(Fchunks, idx1):
    if len(Fchunks) > 3:
        return jnp.concatenate(
            [_sc_gather(Fchunks[:3], idx1), _sc_gather(Fchunks[3:], idx1)],
            axis=-1)
    R = idx1.shape[0]
    nch = len(Fchunks)
    Ptot = 128 * nch
    W = 256 if nch == 1 else 128
    i1 = idx1.reshape(1, R)
    mesh = plsc.VectorSubcoreMesh(core_axis_name="c", subcore_axis_name="s")

    @pl.kernel(out_type=jax.ShapeDtypeStruct((R, Ptot), jnp.float32),
               mesh=mesh)
    def gather_kernel(*refs):
        f_hbms = refs[:nch]
        i1_hbm = refs[nch]
        o_hbm = refs[nch + 1]

        def body(i1_v, o_v):
            for j in range(nch):
                pltpu.sync_copy(f_hbms[j].at[i1_v.at[0]],
                                o_v.at[:, pl.ds(128 * j, 128)])

        pltpu.emit_pipeline(
            body,
            grid=(R // W,),
            in_specs=[pl.BlockSpec((1, W), lambda i: (0, i))],
            out_specs=[pl.BlockSpec((W, Ptot), lambda i: (i, 0))],
            core_axis_name=("c", "s"),
            dimension_semantics=(pltpu.PARALLEL,),
        )(i1_hbm, o_hbm)

    return gather_kernel(*Fchunks, i1)


def _conv1_body(C, K, w_ref, pp_ref, nx_ref, g_ref, y_ref, st_ref):
    g = g_ref[...]
    Rblk, Ptot = g.shape
    ctr = nx_ref[...]
    Cb = ctr.shape[0]
    rr = jax.lax.broadcasted_iota(jnp.int32, (Rblk, Cb), 0) // K
    cc = jax.lax.broadcasted_iota(jnp.int32, (Rblk, Cb), 1)
    E = (rr == cc).astype(jnp.float32)
    crep = jax.lax.dot_general(E, ctr, (((1,), (0,)), ((), ())),
                               preferred_element_type=jnp.float32,
                               precision=jax.lax.Precision.HIGHEST)
    lanepos = C % 128
    chunk = C // 128
    rolled = pltpu.roll(crep, lanepos, axis=1)
    lane = jax.lax.broadcasted_iota(jnp.int32, (Rblk, 128), 1)
    masked = jnp.where((lane >= lanepos) & (lane < lanepos + 3), rolled, 0.0)
    nch = Ptot // 128
    parts = ([jnp.zeros((Rblk, 128 * chunk), jnp.float32)] if chunk else []) \
        + [masked] \
        + ([jnp.zeros((Rblk, 128 * (nch - chunk - 1)), jnp.float32)]
           if nch - chunk - 1 else [])
    corr = jnp.concatenate(parts, axis=1) if len(parts) > 1 else parts[0]
    xin = (g - corr).astype(jnp.bfloat16)
    y = jax.lax.dot_general(xin, w_ref[...], (((1,), (0,)), ((), ())),
                            preferred_element_type=jnp.float32)
    y = y + pp_ref[0:1, :]
    y_ref[...] = y
    ps = jnp.sum(y, axis=0, keepdims=True)
    pq = jnp.sum(y * y, axis=0, keepdims=True)

    @pl.when(pl.program_id(0) == 0)
    def _():
        st_ref[...] = jnp.zeros_like(st_ref)

    st_ref[0:1, :] += ps
    st_ref[1:2, :] += pq


def _norm_conv_body(n, w_ref, ppp_ref, stp_ref, pp_ref, yp_ref, y_ref, st_ref):
    mean = stp_ref[0:1, :] * (1.0 / n)
    var = stp_ref[1:2, :] * (1.0 / n) - mean * mean
    s = ppp_ref[1:2, :] / jnp.sqrt(var + 1e-5)
    t = ppp_ref[2:3, :] - mean * s
    a = jnp.maximum(yp_ref[...] * s + t, 0.0).astype(jnp.bfloat16)
    y = jax.lax.dot_general(a, w_ref[...], (((1,), (0,)), ((), ())),
                            preferred_element_type=jnp.float32)
    y = y + pp_ref[0:1, :]
    y_ref[...] = y
    ps = jnp.sum(y, axis=0, keepdims=True)
    pq = jnp.sum(y * y, axis=0, keepdims=True)

    @pl.when(pl.program_id(0) == 0)
    def _():
        st_ref[...] = jnp.zeros_like(st_ref)

    st_ref[0:1, :] += ps
    st_ref[1:2, :] += pq


def _norm_max_body(n, ppp_ref, stp_ref, y_ref, o_ref):
    mean = stp_ref[0:1, :] * (1.0 / n)
    var = stp_ref[1:2, :] * (1.0 / n) - mean * mean
    s = (ppp_ref[1:2, :] / jnp.sqrt(var + 1e-5))[None]
    t = (ppp_ref[2:3, :] - stp_ref[0:1, :] * (1.0 / n) * s[0])[None]
    a = jnp.maximum(y_ref[...] * s + t, 0.0)
    o_ref[...] = jnp.max(a, axis=1)


def _conv_branch(G, nx_pad, C, Ws, PPs, B, S, K):
    R = G.shape[0]
    Ptot = G.shape[1]
    Rblk = 2048 if Ptot <= 128 else 1024
    grid = (R // Rblk,)
    Cb = Rblk // K

    def call_conv(body, extra_in, yprev, Cout):
        return pl.pallas_call(
            body,
            grid=grid,
            in_specs=[pl.BlockSpec(a.shape, lambda i, nd=a.ndim: (0,) * nd)
                      for a in extra_in]
            + [pl.BlockSpec((Rblk, yprev.shape[1]), lambda i: (i, 0))],
            out_specs=[
                pl.BlockSpec((Rblk, Cout), lambda i: (i, 0)),
                pl.BlockSpec((8, Cout), lambda i: (0, 0)),
            ],
            out_shape=[
                jax.ShapeDtypeStruct((R, Cout), jnp.float32),
                jax.ShapeDtypeStruct((8, Cout), jnp.float32),
            ],
        )(*extra_in, yprev)

    W1, W2, W3 = Ws
    PP1, PP2, PP3 = PPs
    C1 = W1.shape[1]
    y1, st1 = pl.pallas_call(
        functools.partial(_conv1_body, C, K),
        grid=grid,
        in_specs=[
            pl.BlockSpec(W1.shape, lambda i: (0, 0)),
            pl.BlockSpec(PP1.shape, lambda i: (0, 0)),
            pl.BlockSpec((Cb, 128), lambda i: (i, 0)),
            pl.BlockSpec((Rblk, Ptot), lambda i: (i, 0)),
        ],
        out_specs=[
            pl.BlockSpec((Rblk, C1), lambda i: (i, 0)),
            pl.BlockSpec((8, C1), lambda i: (0, 0)),
        ],
        out_shape=[
            jax.ShapeDtypeStruct((R, C1), jnp.float32),
            jax.ShapeDtypeStruct((8, C1), jnp.float32),
        ],
    )(W1, PP1, nx_pad, G)
    y2, st2 = call_conv(functools.partial(_norm_conv_body, float(R)),
                        [W2, PP1, st1, PP2], y1, W2.shape[1])
    y3, st3 = call_conv(functools.partial(_norm_conv_body, float(R)),
                        [W3, PP2, st2, PP3], y2, W3.shape[1])
    C3 = W3.shape[1]
    y3r = y3.reshape(B * S, K, C3)
    Mblk = 64
    out = pl.pallas_call(
        functools.partial(_norm_max_body, float(R)),
        grid=(B * S // Mblk,),
        in_specs=[
            pl.BlockSpec(PP3.shape, lambda i: (0, 0)),
            pl.BlockSpec(st3.shape, lambda i: (0, 0)),
            pl.BlockSpec((Mblk, K, C3), lambda i: (i, 0, 0)),
        ],
        out_specs=pl.BlockSpec((Mblk, C3), lambda i: (i, 0)),
        out_shape=jax.ShapeDtypeStruct((B * S, C3), jnp.float32),
    )(PP3, st3, y3r)
    return out


def _final_kernel(mean_ref, logv_ref, noise_ref, out_ref):
    logv = jnp.clip(logv_ref[...], -30.0, 20.0)
    stdev = jnp.sqrt(jnp.exp(logv))
    out_ref[...] = (mean_ref[...] + stdev * noise_ref[...]) * 0.18215


def _prep_params(params):
    prepped = []
    for li, ((_, _, _, C, mlps), branches) in enumerate(zip(_SA_CFGS, params)):
        P1c = 128 * (-(-(C + 3) // 128))
        pb = []
        for bi, layers in enumerate(branches):
            Ws, PPs = [], []
            for lj, (W, b, gamma, beta) in enumerate(layers):
                Cout, Cin = W.shape
                Cop = _pad16(Cout) if Cout < 16 else Cout
                if lj == 0:
                    Wt = jnp.zeros((P1c, Cop), jnp.float32)
                    Wt = Wt.at[:C + 3, :Cout].set(W.T)
                else:
                    Wt = jnp.zeros((Cin, Cop), jnp.float32)
                    Wt = Wt.at[:, :Cout].set(W.T)
                PP = jnp.zeros((8, Cop), jnp.float32)
                PP = PP.at[0, :Cout].set(b)
                PP = PP.at[1, :Cout].set(gamma)
                PP = PP.at[2, :Cout].set(beta)
                Ws.append(Wt.astype(jnp.bfloat16))
                PPs.append(PP)
            pb.append((Ws, PPs))
        prepped.append(pb)
    return prepped


def kernel(xyz, noise, params):
    B = xyz.shape[0]
    prepped = _prep_params(params)
    l_xyz_b3n = xyz[:, :3, :]
    feats = jnp.transpose(xyz, (0, 2, 1))
    new_xyz = None
    outs = None
    for (cfg, pb) in zip(_SA_CFGS, prepped):
        S, radii, Ks, C, _ = cfg
        N = feats.shape[1]
        xyz_bn3 = jnp.transpose(l_xyz_b3n, (0, 2, 1))
        P1c = 128 * (-(-(C + 3) // 128))
        F = jnp.concatenate(
            [feats, xyz_bn3,
             jnp.zeros((B, N, P1c - (C + 3)), jnp.float32)], axis=-1
        ).reshape(B * N, P1c)
        Fchunks = [F[:, 128 * j:128 * (j + 1)] for j in range(P1c // 128)]
        fps_idx, new_xyz = _fps_pallas(l_xyz_b3n, S)
        nx_pad = jnp.concatenate(
            [new_xyz, jnp.zeros((B, S, 125), jnp.float32)], axis=-1
        ).reshape(B * S, 128)
        gflats = _ballq_pallas(radii, Ks, l_xyz_b3n, new_xyz)
        outs = []
        for bi in range(2):
            K = Ks[bi]
            idx1 = gflats[bi].reshape(B * S * K)
            G = _sc_gather(Fchunks, idx1)
            Ws, PPs = pb[bi]
            outs.append(_conv_branch(G, nx_pad, C, Ws, PPs, B, S, K))
        feats = jnp.concatenate(
            [o.reshape(B, S, o.shape[1]) for o in outs], axis=-1)
        l_xyz_b3n = jnp.transpose(new_xyz, (0, 2, 1))

    S = _SA_CFGS[-1][0]
    mean = jnp.transpose(outs[0].reshape(B, S, -1)[:, :, :4], (0, 2, 1))
    logv = jnp.transpose(outs[1].reshape(B, S, -1)[:, :, :4], (0, 2, 1))
    x = pl.pallas_call(
        _final_kernel,
        out_shape=jax.ShapeDtypeStruct(mean.shape, mean.dtype),
    )(mean, logv, noise)
    return x, l_xyz_b3n
